# baseline (speedup 1.0000x reference)
import functools

import jax
import jax.numpy as jnp
from jax import lax
from jax.experimental import pallas as pl
from jax.experimental.pallas import tpu as pltpu
from jax.experimental.pallas import tpu_sc as plsc

B2 = 32768
D = 64
NW = 32
PER_W = B2 // NW
CHUNK = 256
IDX_W = 128
LMBDA = 0.01


def _sc_gather_score(h2, offh, t2, offt, r2, offr, ent2, rel2):
    mesh = plsc.VectorSubcoreMesh(core_axis_name="c", subcore_axis_name="s")
    rows_w = PER_W // IDX_W
    rows_c = CHUNK // IDX_W

    @functools.partial(
        pl.kernel,
        mesh=mesh,
        compiler_params=pltpu.CompilerParams(
            needs_layout_passes=False, use_tc_tiling_on_sc=True),
        out_type=[
            jax.ShapeDtypeStruct((B2,), jnp.float32),
            jax.ShapeDtypeStruct((NW, 16), jnp.float32),
        ],
        scratch_types=[
            pltpu.VMEM((rows_w, IDX_W), jnp.int32),
            pltpu.VMEM((rows_w, IDX_W), jnp.int32),
            pltpu.VMEM((rows_w, IDX_W), jnp.int32),
            pltpu.VMEM((rows_w, IDX_W), jnp.int32),
            pltpu.VMEM((rows_w, IDX_W), jnp.int32),
            pltpu.VMEM((rows_w, IDX_W), jnp.int32),
            pltpu.VMEM((CHUNK, 2 * D), jnp.float32),
            pltpu.VMEM((CHUNK, 2 * D), jnp.float32),
            pltpu.VMEM((CHUNK, 2 * D), jnp.float32),
            pltpu.VMEM((PER_W,), jnp.float32),
            pltpu.VMEM((16,), jnp.float32),
            pltpu.SemaphoreType.DMA,
        ],
    )
    def sc_kernel(h2_hbm, offh_hbm, t2_hbm, offt_hbm, r2_hbm, offr_hbm,
                  ent_hbm, rel_hbm, dots_hbm, sq_hbm,
                  h2_v, offh_v, t2_v, offt_v, r2_v, offr_v,
                  h_rows, t_rows, r_rows, dots_v, sq_v, sem):
        wid = lax.axis_index("s") * 2 + lax.axis_index("c")
        lane = lax.broadcasted_iota(jnp.int32, (16,), 0)
        base_row = wid * rows_w

        pltpu.sync_copy(h2_hbm.at[pl.ds(base_row, rows_w)], h2_v)
        pltpu.sync_copy(offh_hbm.at[pl.ds(base_row, rows_w)], offh_v)
        pltpu.sync_copy(t2_hbm.at[pl.ds(base_row, rows_w)], t2_v)
        pltpu.sync_copy(offt_hbm.at[pl.ds(base_row, rows_w)], offt_v)
        pltpu.sync_copy(r2_hbm.at[pl.ds(base_row, rows_w)], r2_v)
        pltpu.sync_copy(offr_hbm.at[pl.ds(base_row, rows_w)], offr_v)

        sq_acc = jnp.zeros((16,), jnp.float32)
        for chunk in range(PER_W // CHUNK):
            descs = []
            for k in range(rows_c):
                krow = chunk * rows_c + k
                dst = pl.ds(k * IDX_W, IDX_W)
                descs.append(pltpu.async_copy(
                    ent_hbm.at[h2_v.at[krow]], h_rows.at[dst], sem))
                descs.append(pltpu.async_copy(
                    ent_hbm.at[t2_v.at[krow]], t_rows.at[dst], sem))
                descs.append(pltpu.async_copy(
                    rel_hbm.at[r2_v.at[krow]], r_rows.at[dst], sem))
            for dsc in descs:
                dsc.wait()

            def group_body(g, sq_acc):
                grow = chunk * rows_c + g // 8
                gcol = (g % 8) * 16
                ohv = offh_v[grow, pl.ds(gcol, 16)]
                otv = offt_v[grow, pl.ds(gcol, 16)]
                orv = offr_v[grow, pl.ds(gcol, 16)]
                svec = jnp.zeros((16,), jnp.float32)
                for j in range(16):
                    row = g * 16 + j
                    oh, ot, orr = ohv[j], otv[j], orv[j]
                    acc = None
                    for c in range(4):
                        hv = h_rows[row, pl.ds(oh + c * 16, 16)]
                        tv = t_rows[row, pl.ds(ot + c * 16, 16)]
                        rv = r_rows[row, pl.ds(orr + c * 16, 16)]
                        p = hv * tv * rv
                        acc = p if acc is None else acc + p
                        sq_acc = sq_acc + (hv * hv + tv * tv + rv * rv)
                    svec = jnp.where(lane == j, jnp.sum(acc), svec)
                dots_v[pl.ds(chunk * CHUNK + g * 16, 16)] = svec
                return sq_acc

            sq_acc = lax.fori_loop(0, CHUNK // 16, group_body, sq_acc)
        pltpu.sync_copy(dots_v, dots_hbm.at[pl.ds(wid * PER_W, PER_W)])
        sq_v[...] = sq_acc
        pltpu.sync_copy(sq_v, sq_hbm.at[wid])

    return sc_kernel(h2, offh, t2, offt, r2, offr, ent2, rel2)


_TP = 512


def _tc_transpose_pairs(tab_t, n_rows):
    grid = (n_rows + 2 * _TP - 1) // (2 * _TP)

    def body(lo_ref, hi_ref, out_ref):
        i64 = jnp.asarray(
            lax.broadcasted_iota(jnp.int32, (64, 64), 0)
            == lax.broadcasted_iota(jnp.int32, (64, 64), 1), jnp.float32)
        out_ref[:, 0:64] = lax.dot_general(
            lo_ref[...], i64, (((0,), (0,)), ((), ())),
            preferred_element_type=jnp.float32)
        out_ref[:, 64:128] = lax.dot_general(
            hi_ref[...], i64, (((0,), (0,)), ((), ())),
            preferred_element_type=jnp.float32)

    return pl.pallas_call(
        body,
        grid=(grid,),
        in_specs=[
            pl.BlockSpec((64, _TP), lambda i: (0, 2 * i)),
            pl.BlockSpec((64, _TP), lambda i: (0, 2 * i + 1)),
        ],
        out_specs=pl.BlockSpec((_TP, 128), lambda i: (i, 0)),
        out_shape=jax.ShapeDtypeStruct((grid * _TP, 128), jnp.float32),
    )(tab_t, tab_t)


def _finalize(dots, sq):
    rows = B2 // 128

    def body(dots_ref, sq_ref, out_ref):
        s = dots_ref[...]
        rowid = lax.broadcasted_iota(jnp.int32, (rows, 128), 0)
        x = jnp.where(rowid < rows // 2, -s, s)
        sp = jnp.maximum(x, 0.0) + jnp.log1p(jnp.exp(-jnp.abs(x)))
        mean_sp = jnp.sum(sp) / float(B2)
        regul = jnp.sum(sq_ref[...]) / float(B2 * D)
        out_ref[...] = jnp.reshape(mean_sp + LMBDA * regul, (1, 1))

    return pl.pallas_call(
        body,
        out_shape=jax.ShapeDtypeStruct((1, 1), jnp.float32),
    )(dots.reshape(rows, 128), sq)


def kernel(pos_h, pos_r, pos_t, neg_h, neg_r, neg_t, entity_emb, relation_emb):
    h_idx = jnp.concatenate([pos_h, neg_h])
    t_idx = jnp.concatenate([pos_t, neg_t])
    r_idx = jnp.concatenate([pos_r[:, 0], neg_r[:, 0]])

    def split(idx):
        row = (idx >> 10) * _TP + (idx & (_TP - 1))
        off = ((idx >> 9) & 1) * D
        return (jnp.reshape(row, (B2 // IDX_W, IDX_W)),
                jnp.reshape(off, (B2 // IDX_W, IDX_W)))

    h2, offh = split(h_idx)
    t2, offt = split(t_idx)
    r2, offr = split(r_idx)
    ent2 = _tc_transpose_pairs(entity_emb.T, 1000000)
    rel2 = _tc_transpose_pairs(relation_emb.T, 1000)
    dots, sq = _sc_gather_score(h2, offh, t2, offt, r2, offr, ent2, rel2)
    return _finalize(dots, sq)[0, 0]


# speedup vs baseline: 2.6244x; 2.6244x over previous
import functools

import jax
import jax.numpy as jnp
from jax import lax
from jax.experimental import pallas as pl
from jax.experimental.pallas import tpu as pltpu
from jax.experimental.pallas import tpu_sc as plsc

B2 = 32768
D = 64
NW = 32
PER_W = B2 // NW
CHUNK = 256
IDX_W = 128
LMBDA = 0.01


def _sc_gather_score(h2, offh, t2, offt, r2, offr, ent2, rel2):
    mesh = plsc.VectorSubcoreMesh(core_axis_name="c", subcore_axis_name="s")
    rows_w = PER_W // IDX_W
    rows_c = CHUNK // IDX_W

    @functools.partial(
        pl.kernel,
        mesh=mesh,
        compiler_params=pltpu.CompilerParams(
            needs_layout_passes=False, use_tc_tiling_on_sc=True),
        out_type=[
            jax.ShapeDtypeStruct((B2,), jnp.float32),
            jax.ShapeDtypeStruct((NW, 16), jnp.float32),
        ],
        scratch_types=[
            pltpu.VMEM((rows_w, IDX_W), jnp.int32),
            pltpu.VMEM((rows_w, IDX_W), jnp.int32),
            pltpu.VMEM((rows_w, IDX_W), jnp.int32),
            pltpu.VMEM((rows_w, IDX_W), jnp.int32),
            pltpu.VMEM((rows_w, IDX_W), jnp.int32),
            pltpu.VMEM((rows_w, IDX_W), jnp.int32),
            pltpu.VMEM((CHUNK, 2 * D), jnp.float32),
            pltpu.VMEM((CHUNK, 2 * D), jnp.float32),
            pltpu.VMEM((CHUNK, 2 * D), jnp.float32),
            pltpu.VMEM((PER_W,), jnp.float32),
            pltpu.VMEM((16,), jnp.float32),
            pltpu.SemaphoreType.DMA,
        ],
    )
    def sc_kernel(h2_hbm, offh_hbm, t2_hbm, offt_hbm, r2_hbm, offr_hbm,
                  ent_hbm, rel_hbm, dots_hbm, sq_hbm,
                  h2_v, offh_v, t2_v, offt_v, r2_v, offr_v,
                  h_rows, t_rows, r_rows, dots_v, sq_v, sem):
        wid = lax.axis_index("s") * 2 + lax.axis_index("c")
        lane = lax.broadcasted_iota(jnp.int32, (16,), 0)
        base_row = wid * rows_w

        pltpu.sync_copy(h2_hbm.at[pl.ds(base_row, rows_w)], h2_v)
        pltpu.sync_copy(offh_hbm.at[pl.ds(base_row, rows_w)], offh_v)
        pltpu.sync_copy(t2_hbm.at[pl.ds(base_row, rows_w)], t2_v)
        pltpu.sync_copy(offt_hbm.at[pl.ds(base_row, rows_w)], offt_v)
        pltpu.sync_copy(r2_hbm.at[pl.ds(base_row, rows_w)], r2_v)
        pltpu.sync_copy(offr_hbm.at[pl.ds(base_row, rows_w)], offr_v)

        sq_acc = jnp.zeros((16,), jnp.float32)
        for chunk in range(PER_W // CHUNK):
            descs = []
            for k in range(rows_c):
                krow = chunk * rows_c + k
                dst = pl.ds(k * IDX_W, IDX_W)
                descs.append(pltpu.async_copy(
                    ent_hbm.at[h2_v.at[krow]], h_rows.at[dst], sem))
                descs.append(pltpu.async_copy(
                    ent_hbm.at[t2_v.at[krow]], t_rows.at[dst], sem))
                descs.append(pltpu.async_copy(
                    rel_hbm.at[r2_v.at[krow]], r_rows.at[dst], sem))
            for dsc in descs:
                dsc.wait()

            def group_body(g, sq_acc):
                grow = chunk * rows_c + g // 8
                gcol = (g % 8) * 16
                ohv = offh_v[grow, pl.ds(gcol, 16)]
                otv = offt_v[grow, pl.ds(gcol, 16)]
                orv = offr_v[grow, pl.ds(gcol, 16)]
                svec = jnp.zeros((16,), jnp.float32)
                for j in range(16):
                    row = g * 16 + j
                    oh, ot, orr = ohv[j], otv[j], orv[j]
                    acc = None
                    for c in range(4):
                        hv = h_rows[row, pl.ds(oh + c * 16, 16)]
                        tv = t_rows[row, pl.ds(ot + c * 16, 16)]
                        rv = r_rows[row, pl.ds(orr + c * 16, 16)]
                        p = hv * tv * rv
                        acc = p if acc is None else acc + p
                        sq_acc = sq_acc + (hv * hv + tv * tv + rv * rv)
                    svec = jnp.where(lane == j, jnp.sum(acc), svec)
                dots_v[pl.ds(chunk * CHUNK + g * 16, 16)] = svec
                return sq_acc

            sq_acc = lax.fori_loop(0, CHUNK // 16, group_body, sq_acc)
        pltpu.sync_copy(dots_v, dots_hbm.at[pl.ds(wid * PER_W, PER_W)])
        sq_v[...] = sq_acc
        pltpu.sync_copy(sq_v, sq_hbm.at[wid])

    return sc_kernel(h2, offh, t2, offt, r2, offr, ent2, rel2)


_TP = 8192


def _tc_transpose_pairs(tab_t, n_rows):
    grid = (n_rows + 2 * _TP - 1) // (2 * _TP)

    def body(lo_ref, hi_ref, out_ref):
        i64 = jnp.asarray(
            lax.broadcasted_iota(jnp.int32, (64, 64), 0)
            == lax.broadcasted_iota(jnp.int32, (64, 64), 1), jnp.float32)
        out_ref[:, 0:64] = lax.dot_general(
            lo_ref[...], i64, (((0,), (0,)), ((), ())),
            preferred_element_type=jnp.float32)
        out_ref[:, 64:128] = lax.dot_general(
            hi_ref[...], i64, (((0,), (0,)), ((), ())),
            preferred_element_type=jnp.float32)

    def hi_map(i):
        j = 2 * i + 1
        return (0, jnp.where(j * _TP >= n_rows, 0, j))

    return pl.pallas_call(
        body,
        grid=(grid,),
        in_specs=[
            pl.BlockSpec((64, _TP), lambda i: (0, 2 * i)),
            pl.BlockSpec((64, _TP), hi_map),
        ],
        out_specs=pl.BlockSpec((_TP, 128), lambda i: (i, 0)),
        out_shape=jax.ShapeDtypeStruct((grid * _TP, 128), jnp.float32),
    )(tab_t, tab_t)


def _finalize(dots, sq):
    rows = B2 // 128

    def body(dots_ref, sq_ref, out_ref):
        s = dots_ref[...]
        rowid = lax.broadcasted_iota(jnp.int32, (rows, 128), 0)
        x = jnp.where(rowid < rows // 2, -s, s)
        sp = jnp.maximum(x, 0.0) + jnp.log1p(jnp.exp(-jnp.abs(x)))
        mean_sp = jnp.sum(sp) / float(B2)
        regul = jnp.sum(sq_ref[...]) / float(B2 * D)
        out_ref[...] = jnp.reshape(mean_sp + LMBDA * regul, (1, 1))

    return pl.pallas_call(
        body,
        out_shape=jax.ShapeDtypeStruct((1, 1), jnp.float32),
    )(dots.reshape(rows, 128), sq)


def kernel(pos_h, pos_r, pos_t, neg_h, neg_r, neg_t, entity_emb, relation_emb):
    h_idx = jnp.concatenate([pos_h, neg_h])
    t_idx = jnp.concatenate([pos_t, neg_t])
    r_idx = jnp.concatenate([pos_r[:, 0], neg_r[:, 0]])

    def split(idx):
        row = (idx // (2 * _TP)) * _TP + (idx & (_TP - 1))
        off = ((idx // _TP) & 1) * D
        return (jnp.reshape(row, (B2 // IDX_W, IDX_W)),
                jnp.reshape(off, (B2 // IDX_W, IDX_W)))

    h2, offh = split(h_idx)
    t2, offt = split(t_idx)
    r2, offr = split(r_idx)
    ent2 = _tc_transpose_pairs(entity_emb.T, 1000000)
    rel2 = _tc_transpose_pairs(relation_emb.T, 1000)
    dots, sq = _sc_gather_score(h2, offh, t2, offt, r2, offr, ent2, rel2)
    return _finalize(dots, sq)[0, 0]


# speedup vs baseline: 2.7618x; 1.0523x over previous
import functools

import jax
import jax.numpy as jnp
from jax import lax
from jax.experimental import pallas as pl
from jax.experimental.pallas import tpu as pltpu
from jax.experimental.pallas import tpu_sc as plsc

B2 = 32768
D = 64
NW = 32
PER_W = B2 // NW
CHUNK = 128
IDX_W = 128
LMBDA = 0.01


def _sc_gather_score(h2, offh, t2, offt, r2, offr, ent2, rel2):
    mesh = plsc.VectorSubcoreMesh(core_axis_name="c", subcore_axis_name="s")
    rows_w = PER_W // IDX_W
    rows_c = CHUNK // IDX_W

    @functools.partial(
        pl.kernel,
        mesh=mesh,
        compiler_params=pltpu.CompilerParams(
            needs_layout_passes=False, use_tc_tiling_on_sc=True),
        out_type=[
            jax.ShapeDtypeStruct((B2,), jnp.float32),
            jax.ShapeDtypeStruct((NW, 16), jnp.float32),
        ],
        scratch_types=[
            pltpu.VMEM((rows_w, IDX_W), jnp.int32),
            pltpu.VMEM((rows_w, IDX_W), jnp.int32),
            pltpu.VMEM((rows_w, IDX_W), jnp.int32),
            pltpu.VMEM((rows_w, IDX_W), jnp.int32),
            pltpu.VMEM((rows_w, IDX_W), jnp.int32),
            pltpu.VMEM((rows_w, IDX_W), jnp.int32),
            pltpu.VMEM((CHUNK, 2 * D), jnp.float32),
            pltpu.VMEM((CHUNK, 2 * D), jnp.float32),
            pltpu.VMEM((CHUNK, 2 * D), jnp.float32),
            pltpu.VMEM((CHUNK, 2 * D), jnp.float32),
            pltpu.VMEM((CHUNK, 2 * D), jnp.float32),
            pltpu.VMEM((CHUNK, 2 * D), jnp.float32),
            pltpu.VMEM((PER_W,), jnp.float32),
            pltpu.VMEM((16,), jnp.float32),
            pltpu.SemaphoreType.DMA,
            pltpu.SemaphoreType.DMA,
        ],
    )
    def sc_kernel(h2_hbm, offh_hbm, t2_hbm, offt_hbm, r2_hbm, offr_hbm,
                  ent_hbm, rel_hbm, dots_hbm, sq_hbm,
                  h2_v, offh_v, t2_v, offt_v, r2_v, offr_v,
                  h_rows0, t_rows0, r_rows0, h_rows1, t_rows1, r_rows1,
                  dots_v, sq_v, sem0, sem1):
        wid = lax.axis_index("s") * 2 + lax.axis_index("c")
        lane = lax.broadcasted_iota(jnp.int32, (16,), 0)
        base_row = wid * rows_w

        pltpu.sync_copy(h2_hbm.at[pl.ds(base_row, rows_w)], h2_v)
        pltpu.sync_copy(offh_hbm.at[pl.ds(base_row, rows_w)], offh_v)
        pltpu.sync_copy(t2_hbm.at[pl.ds(base_row, rows_w)], t2_v)
        pltpu.sync_copy(offt_hbm.at[pl.ds(base_row, rows_w)], offt_v)
        pltpu.sync_copy(r2_hbm.at[pl.ds(base_row, rows_w)], r2_v)
        pltpu.sync_copy(offr_hbm.at[pl.ds(base_row, rows_w)], offr_v)

        bufs = [(h_rows0, t_rows0, r_rows0), (h_rows1, t_rows1, r_rows1)]
        sems = [sem0, sem1]
        n_chunks = PER_W // CHUNK

        def issue(chunk):
            hb, tb, rb = bufs[chunk % 2]
            sem = sems[chunk % 2]
            return [
                pltpu.async_copy(ent_hbm.at[h2_v.at[chunk]], hb, sem),
                pltpu.async_copy(ent_hbm.at[t2_v.at[chunk]], tb, sem),
                pltpu.async_copy(rel_hbm.at[r2_v.at[chunk]], rb, sem),
            ]

        sq_acc = jnp.zeros((16,), jnp.float32)
        pending = {0: issue(0)}
        for chunk in range(n_chunks):
            for dsc in pending.pop(chunk):
                dsc.wait()
            if chunk + 1 < n_chunks:
                pending[chunk + 1] = issue(chunk + 1)
            h_rows, t_rows, r_rows = bufs[chunk % 2]

            def group_body(g, sq_acc, chunk=chunk,
                           h_rows=h_rows, t_rows=t_rows, r_rows=r_rows):
                gcol = g * 16
                ohv = offh_v[chunk, pl.ds(gcol, 16)]
                otv = offt_v[chunk, pl.ds(gcol, 16)]
                orv = offr_v[chunk, pl.ds(gcol, 16)]
                svec = jnp.zeros((16,), jnp.float32)
                for j in range(16):
                    row = g * 16 + j
                    oh, ot, orr = ohv[j], otv[j], orv[j]
                    acc = None
                    for c in range(4):
                        hv = h_rows[row, pl.ds(oh + c * 16, 16)]
                        tv = t_rows[row, pl.ds(ot + c * 16, 16)]
                        rv = r_rows[row, pl.ds(orr + c * 16, 16)]
                        p = hv * tv * rv
                        acc = p if acc is None else acc + p
                        sq_acc = sq_acc + (hv * hv + tv * tv + rv * rv)
                    svec = jnp.where(lane == j, jnp.sum(acc), svec)
                dots_v[pl.ds(chunk * CHUNK + g * 16, 16)] = svec
                return sq_acc

            sq_acc = lax.fori_loop(0, CHUNK // 16, group_body, sq_acc)
        pltpu.sync_copy(dots_v, dots_hbm.at[pl.ds(wid * PER_W, PER_W)])
        sq_v[...] = sq_acc
        pltpu.sync_copy(sq_v, sq_hbm.at[wid])

    return sc_kernel(h2, offh, t2, offt, r2, offr, ent2, rel2)


_TP = 8192


def _tc_transpose_pairs(tab_t, n_rows):
    grid = (n_rows + 2 * _TP - 1) // (2 * _TP)

    def body(lo_ref, hi_ref, out_ref):
        out_ref[...] = jnp.concatenate(
            [jnp.transpose(lo_ref[...]), jnp.transpose(hi_ref[...])], axis=1)

    def hi_map(i):
        j = 2 * i + 1
        return (0, jnp.where(j * _TP >= n_rows, 0, j))

    return pl.pallas_call(
        body,
        grid=(grid,),
        in_specs=[
            pl.BlockSpec((64, _TP), lambda i: (0, 2 * i)),
            pl.BlockSpec((64, _TP), hi_map),
        ],
        out_specs=pl.BlockSpec((_TP, 128), lambda i: (i, 0)),
        out_shape=jax.ShapeDtypeStruct((grid * _TP, 128), jnp.float32),
    )(tab_t, tab_t)


def _finalize(dots, sq):
    rows = B2 // 128

    def body(dots_ref, sq_ref, out_ref):
        s = dots_ref[...]
        rowid = lax.broadcasted_iota(jnp.int32, (rows, 128), 0)
        x = jnp.where(rowid < rows // 2, -s, s)
        sp = jnp.maximum(x, 0.0) + jnp.log1p(jnp.exp(-jnp.abs(x)))
        mean_sp = jnp.sum(sp) / float(B2)
        regul = jnp.sum(sq_ref[...]) / float(B2 * D)
        out_ref[...] = jnp.reshape(mean_sp + LMBDA * regul, (1, 1))

    return pl.pallas_call(
        body,
        out_shape=jax.ShapeDtypeStruct((1, 1), jnp.float32),
    )(dots.reshape(rows, 128), sq)


def kernel(pos_h, pos_r, pos_t, neg_h, neg_r, neg_t, entity_emb, relation_emb):
    h_idx = jnp.concatenate([pos_h, neg_h])
    t_idx = jnp.concatenate([pos_t, neg_t])
    r_idx = jnp.concatenate([pos_r[:, 0], neg_r[:, 0]])

    def split(idx):
        row = (idx // (2 * _TP)) * _TP + (idx & (_TP - 1))
        off = ((idx // _TP) & 1) * D
        return (jnp.reshape(row, (B2 // IDX_W, IDX_W)),
                jnp.reshape(off, (B2 // IDX_W, IDX_W)))

    h2, offh = split(h_idx)
    t2, offt = split(t_idx)
    r2, offr = split(r_idx)
    ent2 = _tc_transpose_pairs(entity_emb.T, 1000000)
    rel2 = _tc_transpose_pairs(relation_emb.T, 1000)
    dots, sq = _sc_gather_score(h2, offh, t2, offt, r2, offr, ent2, rel2)
    return _finalize(dots, sq)[0, 0]


# speedup vs baseline: 2.8339x; 1.0261x over previous
import functools

import jax
import jax.numpy as jnp
from jax import lax
from jax.experimental import pallas as pl
from jax.experimental.pallas import tpu as pltpu
from jax.experimental.pallas import tpu_sc as plsc

B2 = 32768
D = 64
NW = 32
PER_W = B2 // NW
CHUNK = 128
IDX_W = 128
LMBDA = 0.01


def _sc_gather_score(h2, offh, t2, offt, r2, offr, ent2, rel2):
    mesh = plsc.VectorSubcoreMesh(core_axis_name="c", subcore_axis_name="s")
    rows_w = PER_W // IDX_W
    rows_c = CHUNK // IDX_W

    @functools.partial(
        pl.kernel,
        mesh=mesh,
        compiler_params=pltpu.CompilerParams(
            needs_layout_passes=False, use_tc_tiling_on_sc=True),
        out_type=[
            jax.ShapeDtypeStruct((B2,), jnp.float32),
            jax.ShapeDtypeStruct((NW, 16), jnp.float32),
        ],
        scratch_types=[
            pltpu.VMEM((rows_w, IDX_W), jnp.int32),
            pltpu.VMEM((rows_w, IDX_W), jnp.int32),
            pltpu.VMEM((rows_w, IDX_W), jnp.int32),
            pltpu.VMEM((rows_w, IDX_W), jnp.int32),
            pltpu.VMEM((rows_w, IDX_W), jnp.int32),
            pltpu.VMEM((rows_w, IDX_W), jnp.int32),
            pltpu.VMEM((CHUNK, 2 * D), jnp.float32),
            pltpu.VMEM((CHUNK, 2 * D), jnp.float32),
            pltpu.VMEM((CHUNK, 2 * D), jnp.float32),
            pltpu.VMEM((CHUNK, 2 * D), jnp.float32),
            pltpu.VMEM((CHUNK, 2 * D), jnp.float32),
            pltpu.VMEM((CHUNK, 2 * D), jnp.float32),
            pltpu.VMEM((PER_W,), jnp.float32),
            pltpu.VMEM((16,), jnp.float32),
            pltpu.SemaphoreType.DMA,
            pltpu.SemaphoreType.DMA,
        ],
    )
    def sc_kernel(h2_hbm, offh_hbm, t2_hbm, offt_hbm, r2_hbm, offr_hbm,
                  ent_hbm, rel_hbm, dots_hbm, sq_hbm,
                  h2_v, offh_v, t2_v, offt_v, r2_v, offr_v,
                  h_rows0, t_rows0, r_rows0, h_rows1, t_rows1, r_rows1,
                  dots_v, sq_v, sem0, sem1):
        wid = lax.axis_index("s") * 2 + lax.axis_index("c")
        lane = lax.broadcasted_iota(jnp.int32, (16,), 0)
        base_row = wid * rows_w

        pltpu.sync_copy(h2_hbm.at[pl.ds(base_row, rows_w)], h2_v)
        pltpu.sync_copy(offh_hbm.at[pl.ds(base_row, rows_w)], offh_v)
        pltpu.sync_copy(t2_hbm.at[pl.ds(base_row, rows_w)], t2_v)
        pltpu.sync_copy(offt_hbm.at[pl.ds(base_row, rows_w)], offt_v)
        pltpu.sync_copy(r2_hbm.at[pl.ds(base_row, rows_w)], r2_v)
        pltpu.sync_copy(offr_hbm.at[pl.ds(base_row, rows_w)], offr_v)

        bufs = [(h_rows0, t_rows0, r_rows0), (h_rows1, t_rows1, r_rows1)]
        sems = [sem0, sem1]
        n_chunks = PER_W // CHUNK

        def issue(chunk):
            hb, tb, rb = bufs[chunk % 2]
            sem = sems[chunk % 2]
            return [
                pltpu.async_copy(ent_hbm.at[h2_v.at[chunk]], hb, sem),
                pltpu.async_copy(ent_hbm.at[t2_v.at[chunk]], tb, sem),
                pltpu.async_copy(rel_hbm.at[r2_v.at[chunk]], rb, sem),
            ]

        sq_acc = jnp.zeros((16,), jnp.float32)
        pending = {0: issue(0)}
        for chunk in range(n_chunks):
            for dsc in pending.pop(chunk):
                dsc.wait()
            if chunk + 1 < n_chunks:
                pending[chunk + 1] = issue(chunk + 1)
            h_rows, t_rows, r_rows = bufs[chunk % 2]

            def group_body(g, sq_acc, chunk=chunk,
                           h_rows=h_rows, t_rows=t_rows, r_rows=r_rows):
                gcol = g * 16
                ohv = offh_v[chunk, pl.ds(gcol, 16)]
                otv = offt_v[chunk, pl.ds(gcol, 16)]
                orv = offr_v[chunk, pl.ds(gcol, 16)]
                svec = jnp.zeros((16,), jnp.float32)
                for j in range(16):
                    row = g * 16 + j
                    oh, ot, orr = ohv[j], otv[j], orv[j]
                    acc = None
                    for c in range(4):
                        hv = h_rows[row, pl.ds(oh + c * 16, 16)]
                        tv = t_rows[row, pl.ds(ot + c * 16, 16)]
                        rv = r_rows[row, pl.ds(orr + c * 16, 16)]
                        p = hv * tv * rv
                        acc = p if acc is None else acc + p
                        sq_acc = sq_acc + (hv * hv + tv * tv + rv * rv)
                    svec = jnp.where(lane == j, jnp.sum(acc), svec)
                dots_v[pl.ds(chunk * CHUNK + g * 16, 16)] = svec
                return sq_acc

            sq_acc = lax.fori_loop(0, CHUNK // 16, group_body, sq_acc)
        pltpu.sync_copy(dots_v, dots_hbm.at[pl.ds(wid * PER_W, PER_W)])
        sq_v[...] = sq_acc
        pltpu.sync_copy(sq_v, sq_hbm.at[wid])

    return sc_kernel(h2, offh, t2, offt, r2, offr, ent2, rel2)


_TP = 12800


def _tc_transpose_pairs(tab_t, n_rows):
    grid = (n_rows + 2 * _TP - 1) // (2 * _TP)

    def body(lo_ref, hi_ref, out_ref):
        out_ref[...] = jnp.concatenate(
            [jnp.transpose(lo_ref[...]), jnp.transpose(hi_ref[...])], axis=1)

    def hi_map(i):
        j = 2 * i + 1
        return (0, jnp.where(j * _TP >= n_rows, 0, j))

    return pl.pallas_call(
        body,
        grid=(grid,),
        in_specs=[
            pl.BlockSpec((64, _TP), lambda i: (0, 2 * i)),
            pl.BlockSpec((64, _TP), hi_map),
        ],
        out_specs=pl.BlockSpec((_TP, 128), lambda i: (i, 0)),
        out_shape=jax.ShapeDtypeStruct((grid * _TP, 128), jnp.float32),
    )(tab_t, tab_t)


def _finalize(dots, sq):
    rows = B2 // 128

    def body(dots_ref, sq_ref, out_ref):
        s = dots_ref[...]
        rowid = lax.broadcasted_iota(jnp.int32, (rows, 128), 0)
        x = jnp.where(rowid < rows // 2, -s, s)
        sp = jnp.maximum(x, 0.0) + jnp.log1p(jnp.exp(-jnp.abs(x)))
        mean_sp = jnp.sum(sp) / float(B2)
        regul = jnp.sum(sq_ref[...]) / float(B2 * D)
        out_ref[...] = jnp.reshape(mean_sp + LMBDA * regul, (1, 1))

    return pl.pallas_call(
        body,
        out_shape=jax.ShapeDtypeStruct((1, 1), jnp.float32),
    )(dots.reshape(rows, 128), sq)


def kernel(pos_h, pos_r, pos_t, neg_h, neg_r, neg_t, entity_emb, relation_emb):
    h_idx = jnp.concatenate([pos_h, neg_h])
    t_idx = jnp.concatenate([pos_t, neg_t])
    r_idx = jnp.concatenate([pos_r[:, 0], neg_r[:, 0]])

    def split(idx):
        row = (idx // (2 * _TP)) * _TP + (idx & (_TP - 1))
        off = ((idx // _TP) & 1) * D
        return (jnp.reshape(row, (B2 // IDX_W, IDX_W)),
                jnp.reshape(off, (B2 // IDX_W, IDX_W)))

    h2, offh = split(h_idx)
    t2, offt = split(t_idx)
    r2, offr = split(r_idx)
    ent2 = _tc_transpose_pairs(entity_emb.T, 1000000)
    rel2 = _tc_transpose_pairs(relation_emb.T, 1000)
    dots, sq = _sc_gather_score(h2, offh, t2, offt, r2, offr, ent2, rel2)
    return _finalize(dots, sq)[0, 0]


# speedup vs baseline: 2.8945x; 1.0214x over previous
import functools

import jax
import jax.numpy as jnp
from jax import lax
from jax.experimental import pallas as pl
from jax.experimental.pallas import tpu as pltpu
from jax.experimental.pallas import tpu_sc as plsc

B2 = 32768
D = 64
NW = 32
PER_W = B2 // NW
CHUNK = 128
IDX_W = 128
LMBDA = 0.01


def _sc_gather_score(h2, offh, t2, offt, r2, offr, ent2, rel2):
    mesh = plsc.VectorSubcoreMesh(core_axis_name="c", subcore_axis_name="s")
    rows_w = PER_W // IDX_W
    rows_c = CHUNK // IDX_W

    @functools.partial(
        pl.kernel,
        mesh=mesh,
        compiler_params=pltpu.CompilerParams(
            needs_layout_passes=False, use_tc_tiling_on_sc=True),
        out_type=[
            jax.ShapeDtypeStruct((B2,), jnp.float32),
            jax.ShapeDtypeStruct((NW, 16), jnp.float32),
        ],
        scratch_types=[
            pltpu.VMEM((rows_w, IDX_W), jnp.int32),
            pltpu.VMEM((rows_w, IDX_W), jnp.int32),
            pltpu.VMEM((rows_w, IDX_W), jnp.int32),
            pltpu.VMEM((rows_w, IDX_W), jnp.int32),
            pltpu.VMEM((rows_w, IDX_W), jnp.int32),
            pltpu.VMEM((rows_w, IDX_W), jnp.int32),
            pltpu.VMEM((CHUNK, 2 * D), jnp.float32),
            pltpu.VMEM((CHUNK, 2 * D), jnp.float32),
            pltpu.VMEM((CHUNK, 2 * D), jnp.float32),
            pltpu.VMEM((CHUNK, 2 * D), jnp.float32),
            pltpu.VMEM((CHUNK, 2 * D), jnp.float32),
            pltpu.VMEM((CHUNK, 2 * D), jnp.float32),
            pltpu.VMEM((PER_W,), jnp.float32),
            pltpu.VMEM((16,), jnp.float32),
            pltpu.SemaphoreType.DMA,
            pltpu.SemaphoreType.DMA,
        ],
    )
    def sc_kernel(h2_hbm, offh_hbm, t2_hbm, offt_hbm, r2_hbm, offr_hbm,
                  ent_hbm, rel_hbm, dots_hbm, sq_hbm,
                  h2_v, offh_v, t2_v, offt_v, r2_v, offr_v,
                  h_rows0, t_rows0, r_rows0, h_rows1, t_rows1, r_rows1,
                  dots_v, sq_v, sem0, sem1):
        wid = lax.axis_index("s") * 2 + lax.axis_index("c")
        lane = lax.broadcasted_iota(jnp.int32, (16,), 0)
        base_row = wid * rows_w

        pltpu.sync_copy(h2_hbm.at[pl.ds(base_row, rows_w)], h2_v)
        pltpu.sync_copy(offh_hbm.at[pl.ds(base_row, rows_w)], offh_v)
        pltpu.sync_copy(t2_hbm.at[pl.ds(base_row, rows_w)], t2_v)
        pltpu.sync_copy(offt_hbm.at[pl.ds(base_row, rows_w)], offt_v)
        pltpu.sync_copy(r2_hbm.at[pl.ds(base_row, rows_w)], r2_v)
        pltpu.sync_copy(offr_hbm.at[pl.ds(base_row, rows_w)], offr_v)

        bufs = [(h_rows0, t_rows0, r_rows0), (h_rows1, t_rows1, r_rows1)]
        sems = [sem0, sem1]
        n_chunks = PER_W // CHUNK

        def issue(chunk):
            hb, tb, rb = bufs[chunk % 2]
            sem = sems[chunk % 2]
            return [
                pltpu.async_copy(ent_hbm.at[h2_v.at[chunk]], hb, sem),
                pltpu.async_copy(ent_hbm.at[t2_v.at[chunk]], tb, sem),
                pltpu.async_copy(rel_hbm.at[r2_v.at[chunk]], rb, sem),
            ]

        sq_acc = jnp.zeros((16,), jnp.float32)
        pending = {0: issue(0)}
        for chunk in range(n_chunks):
            for dsc in pending.pop(chunk):
                dsc.wait()
            if chunk + 1 < n_chunks:
                pending[chunk + 1] = issue(chunk + 1)
            h_rows, t_rows, r_rows = bufs[chunk % 2]

            def group_body(g, sq_acc, chunk=chunk,
                           h_rows=h_rows, t_rows=t_rows, r_rows=r_rows):
                gcol = g * 16
                ohv = offh_v[chunk, pl.ds(gcol, 16)]
                otv = offt_v[chunk, pl.ds(gcol, 16)]
                orv = offr_v[chunk, pl.ds(gcol, 16)]
                svec = jnp.zeros((16,), jnp.float32)
                for j in range(16):
                    row = g * 16 + j
                    oh, ot, orr = ohv[j], otv[j], orv[j]
                    acc = None
                    for c in range(4):
                        hv = h_rows[row, pl.ds(oh + c * 16, 16)]
                        tv = t_rows[row, pl.ds(ot + c * 16, 16)]
                        rv = r_rows[row, pl.ds(orr + c * 16, 16)]
                        p = hv * tv * rv
                        acc = p if acc is None else acc + p
                        sq_acc = sq_acc + (hv * hv + tv * tv + rv * rv)
                    svec = jnp.where(lane == j, jnp.sum(acc), svec)
                dots_v[pl.ds(chunk * CHUNK + g * 16, 16)] = svec
                return sq_acc

            sq_acc = lax.fori_loop(0, CHUNK // 16, group_body, sq_acc)
        pltpu.sync_copy(dots_v, dots_hbm.at[pl.ds(wid * PER_W, PER_W)])
        sq_v[...] = sq_acc
        pltpu.sync_copy(sq_v, sq_hbm.at[wid])

    return sc_kernel(h2, offh, t2, offt, r2, offr, ent2, rel2)


_TP = 12800
_RTP = 512


def _tc_transpose_pairs(tab_t, n_rows, tp):
    grid = (n_rows + 2 * tp - 1) // (2 * tp)

    def body(lo_ref, hi_ref, out_ref):
        out_ref[...] = jnp.concatenate(
            [jnp.transpose(lo_ref[...]), jnp.transpose(hi_ref[...])], axis=1)

    def hi_map(i):
        j = 2 * i + 1
        return (0, jnp.where(j * tp >= n_rows, 0, j))

    return pl.pallas_call(
        body,
        grid=(grid,),
        in_specs=[
            pl.BlockSpec((64, tp), lambda i: (0, 2 * i)),
            pl.BlockSpec((64, tp), hi_map),
        ],
        out_specs=pl.BlockSpec((tp, 128), lambda i: (i, 0)),
        out_shape=jax.ShapeDtypeStruct((grid * tp, 128), jnp.float32),
    )(tab_t, tab_t)


def _finalize(dots, sq):
    rows = B2 // 128

    def body(dots_ref, sq_ref, out_ref):
        s = dots_ref[...]
        rowid = lax.broadcasted_iota(jnp.int32, (rows, 128), 0)
        x = jnp.where(rowid < rows // 2, -s, s)
        sp = jnp.maximum(x, 0.0) + jnp.log1p(jnp.exp(-jnp.abs(x)))
        mean_sp = jnp.sum(sp) / float(B2)
        regul = jnp.sum(sq_ref[...]) / float(B2 * D)
        out_ref[...] = jnp.reshape(mean_sp + LMBDA * regul, (1, 1))

    return pl.pallas_call(
        body,
        out_shape=jax.ShapeDtypeStruct((1, 1), jnp.float32),
    )(dots.reshape(rows, 128), sq)


def kernel(pos_h, pos_r, pos_t, neg_h, neg_r, neg_t, entity_emb, relation_emb):
    h_idx = jnp.concatenate([pos_h, neg_h])
    t_idx = jnp.concatenate([pos_t, neg_t])
    r_idx = jnp.concatenate([pos_r[:, 0], neg_r[:, 0]])

    def split(idx, tp):
        row = (idx // (2 * tp)) * tp + (idx % tp)
        off = ((idx // tp) & 1) * D
        return (jnp.reshape(row, (B2 // IDX_W, IDX_W)),
                jnp.reshape(off, (B2 // IDX_W, IDX_W)))

    h2, offh = split(h_idx, _TP)
    t2, offt = split(t_idx, _TP)
    r2, offr = split(r_idx, _RTP)
    ent2 = _tc_transpose_pairs(entity_emb.T, 1000000, _TP)
    rel2 = _tc_transpose_pairs(relation_emb.T, 1000, _RTP)
    dots, sq = _sc_gather_score(h2, offh, t2, offt, r2, offr, ent2, rel2)
    return _finalize(dots, sq)[0, 0]
